# Initial kernel scaffold; baseline (speedup 1.0000x reference)
import jax
import jax.numpy as jnp
from jax.experimental import pallas as pl


def kernel(x, router_W, router_b, shared_W1, shared_b1, shared_W2, shared_b2, routed_W1, routed_b1, routed_W2, routed_b2):
    raise NotImplementedError("write your pallas kernel here")


# speedup vs baseline: 2.8616x; 2.8616x over previous
import functools

import jax
import jax.numpy as jnp
from jax import lax
from jax.experimental import pallas as pl
from jax.experimental.pallas import tpu as pltpu

EMB = 1024
INTERMED = 1024
N_EXPERTS = 8
N_SHARED = 2
N_ROUTED = 6
K_ROUTED = 2
N_TOK = 4096
TT = 1024
NT = N_TOK // TT


def _moe_body(x_ref, rw_ref, rb_ref, w1_ref, b1_ref, w2_ref, b2_ref,
              out_ref, gd_ref):
    e = pl.program_id(1)
    x = x_ref[...]

    @pl.when(e == 0)
    def _compute_gates():
        logits = lax.dot_general(x, rw_ref[...],
                                 (((1,), (1,)), ((), ())),
                                 preferred_element_type=jnp.float32)
        logits = logits + rb_ref[...]
        col = lax.broadcasted_iota(jnp.int32, (TT, 128), 1)
        valid = col < N_ROUTED
        neg = jnp.float32(-1e30)
        lm = jnp.where(valid, logits, neg)
        m = jnp.max(lm, axis=1, keepdims=True)
        p = jnp.where(valid, jnp.exp(lm - m), 0.0)
        aff = p / jnp.sum(p, axis=1, keepdims=True)
        v1 = jnp.max(aff, axis=1, keepdims=True)
        i1 = jnp.min(jnp.where((aff == v1) & valid, col, 127), axis=1,
                     keepdims=True)
        rem = valid & (col != i1)
        affr = jnp.where(rem, aff, -1.0)
        v2 = jnp.max(affr, axis=1, keepdims=True)
        i2 = jnp.min(jnp.where((affr == v2) & rem, col, 127), axis=1,
                     keepdims=True)
        routed_col = col - N_SHARED
        g_routed = (jnp.where(i1 == routed_col, v1, 0.0)
                    + jnp.where(i2 == routed_col, v2, 0.0))
        gd = jnp.where(col < N_SHARED, 1.0,
                       jnp.where(col < N_EXPERTS, g_routed, 0.0))
        gd_ref[...] = gd

    col = lax.broadcasted_iota(jnp.int32, (TT, 128), 1)
    g = jnp.sum(jnp.where(col == e, gd_ref[...], 0.0), axis=1, keepdims=True)

    h = lax.dot_general(x, w1_ref[0], (((1,), (1,)), ((), ())),
                        preferred_element_type=jnp.float32)
    h = h + b1_ref[0]
    h = 0.5 * h * (1.0 + lax.erf(h * jnp.float32(0.7071067811865476)))
    y = lax.dot_general(h, w2_ref[0], (((1,), (1,)), ((), ())),
                        preferred_element_type=jnp.float32)
    y = (y + b2_ref[0]) * g

    @pl.when(e == 0)
    def _init():
        out_ref[...] = x + y

    @pl.when(e > 0)
    def _acc():
        out_ref[...] += y


@jax.jit
def _moe_dense(x2d, rw_p, rb_p, w1_all, b1_all, w2_all, b2_all):
    return pl.pallas_call(
        _moe_body,
        grid=(NT, N_EXPERTS),
        in_specs=[
            pl.BlockSpec((TT, EMB), lambda t, e: (t, 0)),
            pl.BlockSpec((128, EMB), lambda t, e: (0, 0)),
            pl.BlockSpec((1, 128), lambda t, e: (0, 0)),
            pl.BlockSpec((1, INTERMED, EMB), lambda t, e: (e, 0, 0)),
            pl.BlockSpec((1, 1, INTERMED), lambda t, e: (e, 0, 0)),
            pl.BlockSpec((1, EMB, INTERMED), lambda t, e: (e, 0, 0)),
            pl.BlockSpec((1, 1, EMB), lambda t, e: (e, 0, 0)),
        ],
        out_specs=pl.BlockSpec((TT, EMB), lambda t, e: (t, 0)),
        out_shape=jax.ShapeDtypeStruct((N_TOK, EMB), jnp.float32),
        scratch_shapes=[pltpu.VMEM((TT, 128), jnp.float32)],
        compiler_params=pltpu.CompilerParams(
            dimension_semantics=("arbitrary", "arbitrary")),
    )(x2d, rw_p, rb_p, w1_all, b1_all, w2_all, b2_all)


def kernel(x, router_W, router_b, shared_W1, shared_b1, shared_W2, shared_b2,
           routed_W1, routed_b1, routed_W2, routed_b2):
    B, S, _ = x.shape
    x2d = x.reshape(B * S, EMB)
    rw_p = jnp.zeros((128, EMB), jnp.float32).at[:N_ROUTED].set(router_W)
    rb_p = jnp.zeros((1, 128), jnp.float32).at[0, :N_ROUTED].set(router_b)
    w1_all = jnp.concatenate([shared_W1, routed_W1], axis=0)
    b1_all = jnp.concatenate([shared_b1, routed_b1], axis=0)[:, None, :]
    w2_all = jnp.concatenate([shared_W2, routed_W2], axis=0)
    b2_all = jnp.concatenate([shared_b2, routed_b2], axis=0)[:, None, :]
    out = _moe_dense(x2d, rw_p, rb_p, w1_all, b1_all, w2_all, b2_all)
    return out.reshape(B, S, EMB)
